# Initial kernel scaffold; baseline (speedup 1.0000x reference)
import functools

import jax
import jax.numpy as jnp
from jax import lax
from jax.experimental import pallas as pl
from jax.experimental.pallas import tpu as pltpu
from jax.experimental.pallas import tpu_sc as plsc

N_NODES = 2048
N_EDGES = 32768
EMB = 16

NC = 2
NS = 16
NW = NC * NS
NODES_PER_W = N_NODES // NW
EDGES_PER_W = N_EDGES // NW
CHUNK = 128
NCHUNK = EDGES_PER_W // CHUNK
ROWS_PER_SUB = N_NODES // NS
LANES = 16

_mesh = plsc.VectorSubcoreMesh(
    core_axis_name="c", subcore_axis_name="s", num_cores=NC, num_subcores=NS
)


@functools.partial(
    pl.kernel,
    out_type=[
        jax.ShapeDtypeStruct((N_NODES, EMB), jnp.float32),
        jax.ShapeDtypeStruct((NC, N_NODES, EMB), jnp.float32),
    ],
    mesh=_mesh,
    scratch_types=[
        pltpu.VMEM((N_NODES,), jnp.int32),
        pltpu.VMEM((NODES_PER_W,), jnp.int32),
        pltpu.VMEM((NODES_PER_W, EMB), jnp.float32),
        pltpu.VMEM((NCHUNK, CHUNK), jnp.int32),
        pltpu.VMEM((NCHUNK, CHUNK), jnp.int32),
        pltpu.VMEM((NCHUNK, CHUNK), jnp.int32),
        pltpu.VMEM((CHUNK, EMB), jnp.float32),
        pltpu.VMEM((ROWS_PER_SUB, EMB), jnp.float32),
        pltpu.VMEM_SHARED((N_NODES, EMB), jnp.float32),
        pltpu.SemaphoreType.DMA,
    ],
)
def _sc_stage(nf_hbm, src_hbm, dst_hbm, table_hbm, x_out, agg_out,
              nf_v, nidx_v, x_chunk, src_v, dst_v, gidx_v, rows_v, zero_v,
              agg_sh, sem):
    c = lax.axis_index("c")
    s = lax.axis_index("s")
    wid = s * NC + c

    def _zero_row(r, carry):
        zero_v[r, :] = jnp.zeros((LANES,), jnp.float32)
        return carry
    lax.fori_loop(0, ROWS_PER_SUB, _zero_row, 0)
    pltpu.sync_copy(zero_v, agg_sh.at[pl.ds(s * ROWS_PER_SUB, ROWS_PER_SUB)])

    base_n = wid * NODES_PER_W
    pltpu.sync_copy(nf_hbm.at[pl.ds(base_n, NODES_PER_W)], nidx_v)
    pltpu.async_copy(table_hbm.at[nidx_v], x_chunk, sem).wait()
    pltpu.sync_copy(x_chunk, x_out.at[pl.ds(base_n, NODES_PER_W)])

    pltpu.sync_copy(nf_hbm, nf_v)
    base_e = wid * NCHUNK
    pltpu.sync_copy(src_hbm.at[pl.ds(base_e, NCHUNK)], src_v)
    pltpu.sync_copy(dst_hbm.at[pl.ds(base_e, NCHUNK)], dst_v)

    for j in range(NCHUNK):
        for i in range(CHUNK // LANES):
            sidx = src_v[j, pl.ds(i * LANES, LANES)]
            gidx_v[j, pl.ds(i * LANES, LANES)] = plsc.load_gather(nf_v, [sidx])

    plsc.subcore_barrier()

    for j in range(NCHUNK):
        pltpu.async_copy(table_hbm.at[gidx_v.at[j]], rows_v, sem).wait()
        pltpu.sync_copy(rows_v, agg_sh.at[dst_v.at[j]], add=True)

    plsc.subcore_barrier()

    @pl.when(s == 0)
    def _():
        pltpu.sync_copy(agg_sh, agg_out.at[c])


BLK = 256


def _tc_body(x_ref, agg_ref, wei_ref, bei_ref, wat_ref, bat_ref,
             out_ref, at_ref):
    i = pl.program_id(0)
    h = x_ref[...] + agg_ref[0] + agg_ref[1]
    ei = lax.dot_general(
        h, wei_ref[...], (((1,), (1,)), ((), ())),
        precision=lax.Precision.HIGHEST,
        preferred_element_type=jnp.float32,
    ) + bei_ref[...]
    blk = lax.dynamic_slice(ei, (i * BLK, 0), (BLK, EMB))
    out_ref[...] = lax.dot_general(
        blk, ei, (((1,), (1,)), ((), ())),
        precision=lax.Precision.HIGHEST,
        preferred_element_type=jnp.float32,
    )

    @pl.when(i == 0)
    def _():
        at0 = lax.dot_general(
            h[0:1, :], wat_ref[...], (((1,), (1,)), ((), ())),
            precision=lax.Precision.HIGHEST,
            preferred_element_type=jnp.float32,
        ) + bat_ref[...]
        at_ref[0, 0] = jnp.mean(at0)


def _tc_stage(x, agg, W_ei, b_ei, W_at, b_at):
    return pl.pallas_call(
        _tc_body,
        grid=(N_NODES // BLK,),
        in_specs=[
            pl.BlockSpec((N_NODES, EMB), lambda i: (0, 0)),
            pl.BlockSpec((NC, N_NODES, EMB), lambda i: (0, 0, 0)),
            pl.BlockSpec((EMB, EMB), lambda i: (0, 0)),
            pl.BlockSpec((1, EMB), lambda i: (0, 0)),
            pl.BlockSpec((EMB, EMB), lambda i: (0, 0)),
            pl.BlockSpec((1, EMB), lambda i: (0, 0)),
        ],
        out_specs=[
            pl.BlockSpec((BLK, N_NODES), lambda i: (i, 0)),
            pl.BlockSpec((1, 1), lambda i: (0, 0)),
        ],
        out_shape=[
            jax.ShapeDtypeStruct((N_NODES, N_NODES), jnp.float32),
            jax.ShapeDtypeStruct((1, 1), jnp.float32),
        ],
    )(x, agg, W_ei, b_ei, W_at, b_at)


def kernel(node_feature, batch_ptr, edge_index, node_index, batch_shape,
           emb_table, W_at, b_at, W_ei, b_ei):
    src = edge_index[:, 0].reshape(NW * NCHUNK, CHUNK)
    dst = edge_index[:, 1].reshape(NW * NCHUNK, CHUNK)
    x, agg = _sc_stage(node_feature, src, dst, emb_table)
    edge_actions, action_type = _tc_stage(
        x, agg, W_ei, b_ei.reshape(1, EMB), W_at, b_at.reshape(1, EMB)
    )
    B = batch_shape.shape[0]
    return jnp.concatenate(
        [edge_actions.reshape(B, N_NODES * N_NODES), action_type], axis=-1
    )


# speedup vs baseline: 3.4555x; 3.4555x over previous
import functools

import jax
import jax.numpy as jnp
from jax import lax
from jax.experimental import pallas as pl
from jax.experimental.pallas import tpu as pltpu
from jax.experimental.pallas import tpu_sc as plsc

N_NODES = 2048
N_EDGES = 32768
EMB = 16

NC = 2
NS = 16
NW = NC * NS
NODES_PER_W = N_NODES // NW
EDGES_PER_W = N_EDGES // NW
CHUNK = 128
NCHUNK = EDGES_PER_W // CHUNK
ROWS_PER_SUB = N_NODES // NS
LANES = 16

_mesh = plsc.VectorSubcoreMesh(
    core_axis_name="c", subcore_axis_name="s", num_cores=NC, num_subcores=NS
)


@functools.partial(
    pl.kernel,
    out_type=[
        jax.ShapeDtypeStruct((N_NODES, EMB), jnp.float32),
        jax.ShapeDtypeStruct((NC, N_NODES, EMB), jnp.float32),
    ],
    mesh=_mesh,
    compiler_params=pltpu.CompilerParams(
        needs_layout_passes=False, use_tc_tiling_on_sc=False
    ),
    scratch_types=[
        pltpu.VMEM((N_NODES,), jnp.int32),
        pltpu.VMEM((NODES_PER_W,), jnp.int32),
        pltpu.VMEM((NODES_PER_W, EMB), jnp.float32),
        pltpu.VMEM((NCHUNK, CHUNK), jnp.int32),
        pltpu.VMEM((NCHUNK, CHUNK), jnp.int32),
        pltpu.VMEM((NCHUNK, CHUNK), jnp.int32),
        pltpu.VMEM((CHUNK, EMB), jnp.float32),
        pltpu.VMEM((ROWS_PER_SUB, EMB), jnp.float32),
        pltpu.VMEM_SHARED((N_NODES, EMB), jnp.float32),
        pltpu.SemaphoreType.DMA,
    ],
)
def _sc_stage(nf_hbm, src_hbm, dst_hbm, table_hbm, x_out, agg_out,
              nf_v, nidx_v, x_chunk, src_v, dst_v, gidx_v, rows_v, zero_v,
              agg_sh, sem):
    c = lax.axis_index("c")
    s = lax.axis_index("s")
    wid = s * NC + c

    def _zero_row(r, carry):
        zero_v[r, :] = jnp.zeros((LANES,), jnp.float32)
        return carry
    lax.fori_loop(0, ROWS_PER_SUB, _zero_row, 0)
    pltpu.sync_copy(zero_v, agg_sh.at[pl.ds(s * ROWS_PER_SUB, ROWS_PER_SUB)])

    base_n = wid * NODES_PER_W
    pltpu.sync_copy(nf_hbm.at[pl.ds(base_n, NODES_PER_W)], nidx_v)
    pltpu.async_copy(table_hbm.at[nidx_v], x_chunk, sem).wait()
    pltpu.sync_copy(x_chunk, x_out.at[pl.ds(base_n, NODES_PER_W)])

    pltpu.sync_copy(nf_hbm, nf_v)
    base_e = wid * NCHUNK
    pltpu.sync_copy(src_hbm.at[pl.ds(base_e, NCHUNK)], src_v)
    pltpu.sync_copy(dst_hbm.at[pl.ds(base_e, NCHUNK)], dst_v)

    for j in range(NCHUNK):
        for i in range(CHUNK // LANES):
            sidx = src_v[j, pl.ds(i * LANES, LANES)]
            gidx_v[j, pl.ds(i * LANES, LANES)] = plsc.load_gather(nf_v, [sidx])

    plsc.subcore_barrier()

    for j in range(NCHUNK):
        pltpu.async_copy(table_hbm.at[gidx_v.at[j]], rows_v, sem).wait()
        pltpu.sync_copy(rows_v, agg_sh.at[dst_v.at[j]], add=True)

    plsc.subcore_barrier()

    @pl.when(s == 0)
    def _():
        pltpu.sync_copy(agg_sh, agg_out.at[c])


BLK = 256


def _tc_body(x_ref, agg_ref, xb_ref, aggb_ref, wei_ref, bei_ref, wat_ref,
             bat_ref, out_ref, at_ref):
    i = pl.program_id(0)
    h = x_ref[...] + agg_ref[0] + agg_ref[1]
    ei = lax.dot_general(
        h, wei_ref[...], (((1,), (1,)), ((), ())),
        precision=lax.Precision.HIGHEST,
        preferred_element_type=jnp.float32,
    ) + bei_ref[...]
    h_blk = xb_ref[...] + aggb_ref[0] + aggb_ref[1]
    ei_blk = lax.dot_general(
        h_blk, wei_ref[...], (((1,), (1,)), ((), ())),
        precision=lax.Precision.HIGHEST,
        preferred_element_type=jnp.float32,
    ) + bei_ref[...]
    out_ref[...] = lax.dot_general(
        ei_blk, ei, (((1,), (1,)), ((), ())),
        precision=lax.Precision.HIGHEST,
        preferred_element_type=jnp.float32,
    )

    @pl.when(i == 0)
    def _():
        at0 = lax.dot_general(
            h[0:1, :], wat_ref[...], (((1,), (1,)), ((), ())),
            precision=lax.Precision.HIGHEST,
            preferred_element_type=jnp.float32,
        ) + bat_ref[...]
        at_ref[...] = jnp.mean(at0, axis=-1, keepdims=True)


def _tc_stage(x, agg, W_ei, b_ei, W_at, b_at):
    return pl.pallas_call(
        _tc_body,
        grid=(N_NODES // BLK,),
        in_specs=[
            pl.BlockSpec((N_NODES, EMB), lambda i: (0, 0)),
            pl.BlockSpec((NC, N_NODES, EMB), lambda i: (0, 0, 0)),
            pl.BlockSpec((BLK, EMB), lambda i: (i, 0)),
            pl.BlockSpec((NC, BLK, EMB), lambda i: (0, i, 0)),
            pl.BlockSpec((EMB, EMB), lambda i: (0, 0)),
            pl.BlockSpec((1, EMB), lambda i: (0, 0)),
            pl.BlockSpec((EMB, EMB), lambda i: (0, 0)),
            pl.BlockSpec((1, EMB), lambda i: (0, 0)),
        ],
        out_specs=[
            pl.BlockSpec((BLK, N_NODES), lambda i: (i, 0)),
            pl.BlockSpec((1, 1), lambda i: (0, 0)),
        ],
        out_shape=[
            jax.ShapeDtypeStruct((N_NODES, N_NODES), jnp.float32),
            jax.ShapeDtypeStruct((1, 1), jnp.float32),
        ],
    )(x, agg, x, agg, W_ei, b_ei, W_at, b_at)


def kernel(node_feature, batch_ptr, edge_index, node_index, batch_shape,
           emb_table, W_at, b_at, W_ei, b_ei):
    src = edge_index[:, 0].reshape(NW * NCHUNK, CHUNK)
    dst = edge_index[:, 1].reshape(NW * NCHUNK, CHUNK)
    x, agg = _sc_stage(node_feature, src, dst, emb_table)
    edge_actions, action_type = _tc_stage(
        x, agg, W_ei, b_ei.reshape(1, EMB), W_at, b_at.reshape(1, EMB)
    )
    B = batch_shape.shape[0]
    return jnp.concatenate(
        [edge_actions.reshape(B, N_NODES * N_NODES), action_type], axis=-1
    )


# speedup vs baseline: 6.8353x; 1.9781x over previous
import functools

import jax
import jax.numpy as jnp
from jax import lax
from jax.experimental import pallas as pl
from jax.experimental.pallas import tpu as pltpu
from jax.experimental.pallas import tpu_sc as plsc

N_NODES = 2048
N_EDGES = 32768
EMB = 16

NC = 2
NS = 16
NW = NC * NS
NODES_PER_W = N_NODES // NW
EDGES_PER_W = N_EDGES // NW
CHUNK = 128
NCHUNK = EDGES_PER_W // CHUNK
ROWS_PER_SUB = N_NODES // NS
LANES = 16

_mesh = plsc.VectorSubcoreMesh(
    core_axis_name="c", subcore_axis_name="s", num_cores=NC, num_subcores=NS
)


@functools.partial(
    pl.kernel,
    out_type=[
        jax.ShapeDtypeStruct((N_NODES, EMB), jnp.float32),
        jax.ShapeDtypeStruct((NC, N_NODES, EMB), jnp.float32),
    ],
    mesh=_mesh,
    compiler_params=pltpu.CompilerParams(
        needs_layout_passes=False, use_tc_tiling_on_sc=False
    ),
    scratch_types=[
        pltpu.VMEM((N_NODES,), jnp.int32),
        pltpu.VMEM((NODES_PER_W,), jnp.int32),
        pltpu.VMEM((NODES_PER_W, EMB), jnp.float32),
        pltpu.VMEM((NCHUNK, CHUNK), jnp.int32),
        pltpu.VMEM((NCHUNK, CHUNK), jnp.int32),
        pltpu.VMEM((NCHUNK, CHUNK), jnp.int32),
        pltpu.VMEM((CHUNK, EMB), jnp.float32),
        pltpu.VMEM((ROWS_PER_SUB, EMB), jnp.float32),
        pltpu.VMEM_SHARED((N_NODES, EMB), jnp.float32),
        pltpu.SemaphoreType.DMA,
    ],
)
def _sc_stage(nf_hbm, src_hbm, dst_hbm, table_hbm, x_out, agg_out,
              nf_v, nidx_v, x_chunk, src_v, dst_v, gidx_v, rows_v, zero_v,
              agg_sh, sem):
    c = lax.axis_index("c")
    s = lax.axis_index("s")
    wid = s * NC + c

    def _zero_row(r, carry):
        zero_v[r, :] = jnp.zeros((LANES,), jnp.float32)
        return carry
    lax.fori_loop(0, ROWS_PER_SUB, _zero_row, 0)
    pltpu.sync_copy(zero_v, agg_sh.at[pl.ds(s * ROWS_PER_SUB, ROWS_PER_SUB)])

    base_n = wid * NODES_PER_W
    pltpu.sync_copy(nf_hbm.at[pl.ds(base_n, NODES_PER_W)], nidx_v)
    pltpu.async_copy(table_hbm.at[nidx_v], x_chunk, sem).wait()
    pltpu.sync_copy(x_chunk, x_out.at[pl.ds(base_n, NODES_PER_W)])

    pltpu.sync_copy(nf_hbm, nf_v)
    base_e = wid * NCHUNK
    pltpu.sync_copy(src_hbm.at[pl.ds(base_e, NCHUNK)], src_v)
    pltpu.sync_copy(dst_hbm.at[pl.ds(base_e, NCHUNK)], dst_v)

    for j in range(NCHUNK):
        for i in range(CHUNK // LANES):
            sidx = src_v[j, pl.ds(i * LANES, LANES)]
            gidx_v[j, pl.ds(i * LANES, LANES)] = plsc.load_gather(nf_v, [sidx])

    plsc.subcore_barrier()

    for j in range(NCHUNK):
        pltpu.async_copy(table_hbm.at[gidx_v.at[j]], rows_v, sem).wait()
        pltpu.sync_copy(rows_v, agg_sh.at[dst_v.at[j]], add=True)

    plsc.subcore_barrier()

    @pl.when(s == 0)
    def _():
        pltpu.sync_copy(agg_sh, agg_out.at[c])


BLK = 256


def _tc_body(x_ref, agg_ref, xb_ref, aggb_ref, wei_ref, bei_ref, wat_ref,
             bat_ref, out_ref):
    i = pl.program_id(0)
    h = x_ref[...] + agg_ref[0] + agg_ref[1]
    ei = lax.dot_general(
        h, wei_ref[...], (((1,), (1,)), ((), ())),
        precision=lax.Precision.HIGHEST,
        preferred_element_type=jnp.float32,
    ) + bei_ref[...]

    @pl.when(i < N_NODES // BLK)
    def _():
        h_blk = xb_ref[...] + aggb_ref[0] + aggb_ref[1]
        ei_blk = lax.dot_general(
            h_blk, wei_ref[...], (((1,), (1,)), ((), ())),
            precision=lax.Precision.HIGHEST,
            preferred_element_type=jnp.float32,
        ) + bei_ref[...]
        gram = lax.dot_general(
            ei_blk, ei, (((1,), (1,)), ((), ())),
            precision=lax.Precision.HIGHEST,
            preferred_element_type=jnp.float32,
        )
        out_ref[...] = gram.reshape(1, BLK * N_NODES)

    @pl.when(i == N_NODES // BLK)
    def _():
        at0 = lax.dot_general(
            h[0:1, :], wat_ref[...], (((1,), (1,)), ((), ())),
            precision=lax.Precision.HIGHEST,
            preferred_element_type=jnp.float32,
        ) + bat_ref[...]
        at = jnp.mean(at0, axis=-1, keepdims=True)
        out_ref[...] = jnp.broadcast_to(at, (1, BLK * N_NODES))


def _tc_stage(x, agg, W_ei, b_ei, W_at, b_at):
    nblk = N_NODES // BLK
    return pl.pallas_call(
        _tc_body,
        grid=(nblk + 1,),
        in_specs=[
            pl.BlockSpec((N_NODES, EMB), lambda i: (0, 0)),
            pl.BlockSpec((NC, N_NODES, EMB), lambda i: (0, 0, 0)),
            pl.BlockSpec((BLK, EMB), lambda i: (i % (N_NODES // BLK), 0)),
            pl.BlockSpec((NC, BLK, EMB),
                         lambda i: (0, i % (N_NODES // BLK), 0)),
            pl.BlockSpec((EMB, EMB), lambda i: (0, 0)),
            pl.BlockSpec((1, EMB), lambda i: (0, 0)),
            pl.BlockSpec((EMB, EMB), lambda i: (0, 0)),
            pl.BlockSpec((1, EMB), lambda i: (0, 0)),
        ],
        out_specs=[
            pl.BlockSpec((1, BLK * N_NODES), lambda i: (0, i)),
        ],
        out_shape=[
            jax.ShapeDtypeStruct((1, N_NODES * N_NODES + 1), jnp.float32),
        ],
    )(x, agg, x, agg, W_ei, b_ei, W_at, b_at)


def kernel(node_feature, batch_ptr, edge_index, node_index, batch_shape,
           emb_table, W_at, b_at, W_ei, b_ei):
    src = edge_index[:, 0].reshape(NW * NCHUNK, CHUNK)
    dst = edge_index[:, 1].reshape(NW * NCHUNK, CHUNK)
    x, agg = _sc_stage(node_feature, src, dst, emb_table)
    (out,) = _tc_stage(
        x, agg, W_ei, b_ei.reshape(1, EMB), W_at, b_at.reshape(1, EMB)
    )
    return out


# speedup vs baseline: 9.0291x; 1.3210x over previous
import functools

import jax
import jax.numpy as jnp
from jax import lax
from jax.experimental import pallas as pl
from jax.experimental.pallas import tpu as pltpu
from jax.experimental.pallas import tpu_sc as plsc

N_NODES = 2048
N_EDGES = 32768
EMB = 16

NC = 2
NS = 16
NW = NC * NS
NODES_PER_W = N_NODES // NW
EDGES_PER_W = N_EDGES // NW
CHUNK = 128
NCHUNK = EDGES_PER_W // CHUNK
ROWS_PER_SUB = N_NODES // NS
LANES = 16

_mesh = plsc.VectorSubcoreMesh(
    core_axis_name="c", subcore_axis_name="s", num_cores=NC, num_subcores=NS
)


@functools.partial(
    pl.kernel,
    out_type=[
        jax.ShapeDtypeStruct((N_NODES, EMB), jnp.float32),
        jax.ShapeDtypeStruct((NC, N_NODES, EMB), jnp.float32),
    ],
    mesh=_mesh,
    compiler_params=pltpu.CompilerParams(
        needs_layout_passes=False, use_tc_tiling_on_sc=False
    ),
    scratch_types=[
        pltpu.VMEM((N_NODES,), jnp.int32),
        pltpu.VMEM((NODES_PER_W,), jnp.int32),
        pltpu.VMEM((NODES_PER_W, EMB), jnp.float32),
        pltpu.VMEM((NCHUNK, CHUNK), jnp.int32),
        pltpu.VMEM((NCHUNK, CHUNK), jnp.int32),
        pltpu.VMEM((NCHUNK, CHUNK), jnp.int32),
        pltpu.VMEM((CHUNK, EMB), jnp.float32),
        pltpu.VMEM((ROWS_PER_SUB, EMB), jnp.float32),
        pltpu.VMEM_SHARED((N_NODES, EMB), jnp.float32),
        pltpu.SemaphoreType.DMA,
    ],
)
def _sc_stage(nf_hbm, src_hbm, dst_hbm, table_hbm, x_out, agg_out,
              nf_v, nidx_v, x_chunk, src_v, dst_v, gidx_v, rows_v, zero_v,
              agg_sh, sem):
    c = lax.axis_index("c")
    s = lax.axis_index("s")
    wid = s * NC + c

    def _zero_row(r, carry):
        zero_v[r, :] = jnp.zeros((LANES,), jnp.float32)
        return carry
    lax.fori_loop(0, ROWS_PER_SUB, _zero_row, 0)
    pltpu.sync_copy(zero_v, agg_sh.at[pl.ds(s * ROWS_PER_SUB, ROWS_PER_SUB)])

    base_n = wid * NODES_PER_W
    pltpu.sync_copy(nf_hbm.at[pl.ds(base_n, NODES_PER_W)], nidx_v)
    pltpu.async_copy(table_hbm.at[nidx_v], x_chunk, sem).wait()
    pltpu.sync_copy(x_chunk, x_out.at[pl.ds(base_n, NODES_PER_W)])

    pltpu.sync_copy(nf_hbm, nf_v)
    base_e = wid * NCHUNK
    pltpu.sync_copy(src_hbm.at[pl.ds(base_e, NCHUNK)], src_v)
    pltpu.sync_copy(dst_hbm.at[pl.ds(base_e, NCHUNK)], dst_v)

    for j in range(NCHUNK):
        for i in range(CHUNK // LANES):
            sidx = src_v[j, pl.ds(i * LANES, LANES)]
            gidx_v[j, pl.ds(i * LANES, LANES)] = plsc.load_gather(nf_v, [sidx])

    plsc.subcore_barrier()

    for j in range(NCHUNK):
        pltpu.async_copy(table_hbm.at[gidx_v.at[j]], rows_v, sem).wait()
        pltpu.sync_copy(rows_v, agg_sh.at[dst_v.at[j]], add=True)

    plsc.subcore_barrier()

    @pl.when(s == 0)
    def _():
        pltpu.sync_copy(agg_sh, agg_out.at[c])


BLK = 256


def _tc_body(x_ref, agg_ref, xb_ref, aggb_ref, wei_ref, bei_ref, wat_ref,
             bat_ref, out_ref):
    i = pl.program_id(0)
    h = x_ref[...] + agg_ref[0] + agg_ref[1]
    ei = lax.dot_general(
        h, wei_ref[...], (((1,), (1,)), ((), ())),
        preferred_element_type=jnp.float32,
    ) + bei_ref[...]

    @pl.when(i < N_NODES // BLK)
    def _():
        h_blk = xb_ref[...] + aggb_ref[0] + aggb_ref[1]
        ei_blk = lax.dot_general(
            h_blk, wei_ref[...], (((1,), (1,)), ((), ())),
                preferred_element_type=jnp.float32,
        ) + bei_ref[...]
        gram = lax.dot_general(
            ei_blk, ei, (((1,), (1,)), ((), ())),
                preferred_element_type=jnp.float32,
        )
        out_ref[...] = gram.reshape(1, BLK * N_NODES)

    @pl.when(i == N_NODES // BLK)
    def _():
        at0 = lax.dot_general(
            h[0:1, :], wat_ref[...], (((1,), (1,)), ((), ())),
                preferred_element_type=jnp.float32,
        ) + bat_ref[...]
        at = jnp.mean(at0, axis=-1, keepdims=True)
        out_ref[...] = jnp.broadcast_to(at, (1, BLK * N_NODES))


def _tc_stage(x, agg, W_ei, b_ei, W_at, b_at):
    nblk = N_NODES // BLK
    return pl.pallas_call(
        _tc_body,
        grid=(nblk + 1,),
        in_specs=[
            pl.BlockSpec((N_NODES, EMB), lambda i: (0, 0)),
            pl.BlockSpec((NC, N_NODES, EMB), lambda i: (0, 0, 0)),
            pl.BlockSpec((BLK, EMB), lambda i: (i % (N_NODES // BLK), 0)),
            pl.BlockSpec((NC, BLK, EMB),
                         lambda i: (0, i % (N_NODES // BLK), 0)),
            pl.BlockSpec((EMB, EMB), lambda i: (0, 0)),
            pl.BlockSpec((1, EMB), lambda i: (0, 0)),
            pl.BlockSpec((EMB, EMB), lambda i: (0, 0)),
            pl.BlockSpec((1, EMB), lambda i: (0, 0)),
        ],
        out_specs=[
            pl.BlockSpec((1, BLK * N_NODES), lambda i: (0, i)),
        ],
        out_shape=[
            jax.ShapeDtypeStruct((1, N_NODES * N_NODES + 1), jnp.float32),
        ],
    )(x, agg, x, agg, W_ei, b_ei, W_at, b_at)


def kernel(node_feature, batch_ptr, edge_index, node_index, batch_shape,
           emb_table, W_at, b_at, W_ei, b_ei):
    src = edge_index[:, 0].reshape(NW * NCHUNK, CHUNK)
    dst = edge_index[:, 1].reshape(NW * NCHUNK, CHUNK)
    x, agg = _sc_stage(node_feature, src, dst, emb_table)
    (out,) = _tc_stage(
        x, agg, W_ei, b_ei.reshape(1, EMB), W_at, b_at.reshape(1, EMB)
    )
    return out


# speedup vs baseline: 11.8844x; 1.3162x over previous
import functools

import jax
import jax.numpy as jnp
from jax import lax
from jax.experimental import pallas as pl
from jax.experimental.pallas import tpu as pltpu
from jax.experimental.pallas import tpu_sc as plsc

N_NODES = 2048
N_EDGES = 32768
EMB = 16

NC = 2
NS = 16
NW = NC * NS
NODES_PER_W = N_NODES // NW
EDGES_PER_W = N_EDGES // NW
CHUNK = 128
NCHUNK = EDGES_PER_W // CHUNK
ROWS_PER_SUB = N_NODES // NS
LANES = 16

_mesh = plsc.VectorSubcoreMesh(
    core_axis_name="c", subcore_axis_name="s", num_cores=NC, num_subcores=NS
)


@functools.partial(
    pl.kernel,
    out_type=jax.ShapeDtypeStruct((NC, N_NODES, EMB), jnp.float32),
    mesh=_mesh,
    compiler_params=pltpu.CompilerParams(
        needs_layout_passes=False, use_tc_tiling_on_sc=False
    ),
    scratch_types=[
        pltpu.VMEM((N_NODES,), jnp.int32),
        pltpu.VMEM((NODES_PER_W,), jnp.int32),
        pltpu.VMEM((NODES_PER_W, EMB), jnp.float32),
        pltpu.VMEM((NCHUNK, CHUNK), jnp.int32),
        pltpu.VMEM((NCHUNK, CHUNK), jnp.int32),
        pltpu.VMEM((NCHUNK, CHUNK), jnp.int32),
        pltpu.VMEM((EDGES_PER_W, EMB), jnp.float32),
        pltpu.VMEM((ROWS_PER_SUB, EMB), jnp.float32),
        pltpu.VMEM_SHARED((N_NODES, EMB), jnp.float32),
        pltpu.SemaphoreType.DMA,
        pltpu.SemaphoreType.DMA,
        pltpu.SemaphoreType.DMA,
        pltpu.SemaphoreType.DMA,
    ],
)
def _sc_stage(nf_hbm, src_hbm, dst_hbm, table_hbm, agg_out,
              nf_v, nid_v, x_chunk, src_v, dst_v, gidx_v, rows_v, zero_v,
              agg_sh, sem_in, sem_x, sem_rows, sem_sc):
    c = lax.axis_index("c")
    s = lax.axis_index("s")
    wid = s * NC + c
    base_n = wid * NODES_PER_W
    base_e = wid * NCHUNK

    cp_nf = pltpu.async_copy(nf_hbm, nf_v, sem_in)
    cp_src = pltpu.async_copy(src_hbm.at[pl.ds(base_e, NCHUNK)], src_v, sem_in)
    cp_dst = pltpu.async_copy(dst_hbm.at[pl.ds(base_e, NCHUNK)], dst_v, sem_in)

    def _zero_row(r, carry):
        zero_v[r, :] = jnp.zeros((LANES,), jnp.float32)
        return carry
    lax.fori_loop(0, ROWS_PER_SUB, _zero_row, 0)
    pltpu.sync_copy(zero_v, agg_sh.at[pl.ds(s * ROWS_PER_SUB, ROWS_PER_SUB)])

    for k in range(NODES_PER_W // LANES):
        nid_v[pl.ds(k * LANES, LANES)] = (
            base_n + k * LANES + lax.broadcasted_iota(jnp.int32, (LANES,), 0)
        )

    cp_nf.wait()
    cp_src.wait()
    cp_dst.wait()

    cp_x = pltpu.async_copy(
        table_hbm.at[nf_v.at[pl.ds(base_n, NODES_PER_W)]], x_chunk, sem_x
    )

    for j in range(NCHUNK):
        for i in range(CHUNK // LANES):
            sidx = src_v[j, pl.ds(i * LANES, LANES)]
            gidx_v[j, pl.ds(i * LANES, LANES)] = plsc.load_gather(nf_v, [sidx])

    row_cps = [
        pltpu.async_copy(
            table_hbm.at[gidx_v.at[j]],
            rows_v.at[pl.ds(j * CHUNK, CHUNK)],
            sem_rows,
        )
        for j in range(NCHUNK)
    ]

    plsc.subcore_barrier()

    cp_x.wait()
    sc_x = pltpu.async_copy(x_chunk, agg_sh.at[nid_v], sem_sc, add=True)
    for cp in row_cps:
        cp.wait()
    sc_cps = [
        pltpu.async_copy(
            rows_v.at[pl.ds(j * CHUNK, CHUNK)],
            agg_sh.at[dst_v.at[j]],
            sem_sc,
            add=True,
        )
        for j in range(NCHUNK)
    ]
    sc_x.wait()
    for cp in sc_cps:
        cp.wait()

    plsc.subcore_barrier()

    @pl.when(s == 0)
    def _():
        pltpu.sync_copy(agg_sh, agg_out.at[c])


BLK = 256
NBLK = N_NODES // BLK


def _tc_body(agg_ref, wei_ref, bei_ref, wat_ref, bat_ref, out_ref, ei_s):
    i = pl.program_id(0)

    @pl.when(i == 0)
    def _():
        h = agg_ref[0] + agg_ref[1]
        ei_s[...] = lax.dot_general(
            h, wei_ref[...], (((1,), (1,)), ((), ())),
            preferred_element_type=jnp.float32,
        ) + bei_ref[...]

    @pl.when(i < NBLK)
    def _():
        ei = ei_s[...]
        ei_blk = ei_s[pl.ds(i * BLK, BLK), :]
        gram = lax.dot_general(
            ei_blk, ei, (((1,), (1,)), ((), ())),
            preferred_element_type=jnp.float32,
        )
        out_ref[...] = gram.reshape(1, BLK * N_NODES)

    @pl.when(i == NBLK)
    def _():
        h0 = agg_ref[0, 0:1, :] + agg_ref[1, 0:1, :]
        at0 = lax.dot_general(
            h0, wat_ref[...], (((1,), (1,)), ((), ())),
            preferred_element_type=jnp.float32,
        ) + bat_ref[...]
        at = jnp.mean(at0, axis=-1, keepdims=True)
        out_ref[...] = jnp.broadcast_to(at, (1, BLK * N_NODES))


def _tc_stage(agg, W_ei, b_ei, W_at, b_at):
    return pl.pallas_call(
        _tc_body,
        grid=(NBLK + 1,),
        in_specs=[
            pl.BlockSpec((NC, N_NODES, EMB), lambda i: (0, 0, 0)),
            pl.BlockSpec((EMB, EMB), lambda i: (0, 0)),
            pl.BlockSpec((1, EMB), lambda i: (0, 0)),
            pl.BlockSpec((EMB, EMB), lambda i: (0, 0)),
            pl.BlockSpec((1, EMB), lambda i: (0, 0)),
        ],
        out_specs=pl.BlockSpec((1, BLK * N_NODES), lambda i: (0, i)),
        out_shape=jax.ShapeDtypeStruct((1, N_NODES * N_NODES + 1),
                                       jnp.float32),
        scratch_shapes=[pltpu.VMEM((N_NODES, EMB), jnp.float32)],
    )(agg, W_ei, b_ei, W_at, b_at)


def kernel(node_feature, batch_ptr, edge_index, node_index, batch_shape,
           emb_table, W_at, b_at, W_ei, b_ei):
    src = edge_index[:, 0].reshape(NW * NCHUNK, CHUNK)
    dst = edge_index[:, 1].reshape(NW * NCHUNK, CHUNK)
    agg = _sc_stage(node_feature, src, dst, emb_table)
    return _tc_stage(
        agg, W_ei, b_ei.reshape(1, EMB), W_at, b_at.reshape(1, EMB)
    )


# speedup vs baseline: 12.2199x; 1.0282x over previous
import functools

import jax
import jax.numpy as jnp
from jax import lax
from jax.experimental import pallas as pl
from jax.experimental.pallas import tpu as pltpu
from jax.experimental.pallas import tpu_sc as plsc

N_NODES = 2048
N_EDGES = 32768
EMB = 16

NC = 2
NS = 16
NW = NC * NS
NODES_PER_W = N_NODES // NW
EDGES_PER_W = N_EDGES // NW
CHUNK = 128
NCHUNK = EDGES_PER_W // CHUNK
ROWS_PER_SUB = N_NODES // NS
LANES = 16

_mesh = plsc.VectorSubcoreMesh(
    core_axis_name="c", subcore_axis_name="s", num_cores=NC, num_subcores=NS
)


@functools.partial(
    pl.kernel,
    out_type=jax.ShapeDtypeStruct((NC, N_NODES, EMB), jnp.float32),
    mesh=_mesh,
    compiler_params=pltpu.CompilerParams(
        needs_layout_passes=False, use_tc_tiling_on_sc=False
    ),
    scratch_types=[
        pltpu.VMEM((N_NODES,), jnp.int32),
        pltpu.VMEM((NODES_PER_W,), jnp.int32),
        pltpu.VMEM((NODES_PER_W, EMB), jnp.float32),
        pltpu.VMEM((NCHUNK, CHUNK), jnp.int32),
        pltpu.VMEM((NCHUNK, CHUNK), jnp.int32),
        pltpu.VMEM((NCHUNK, CHUNK), jnp.int32),
        pltpu.VMEM((EDGES_PER_W, EMB), jnp.float32),
        pltpu.VMEM((ROWS_PER_SUB, EMB), jnp.float32),
        pltpu.VMEM_SHARED((N_NODES, EMB), jnp.float32),
        pltpu.SemaphoreType.DMA,
        pltpu.SemaphoreType.DMA,
        pltpu.SemaphoreType.DMA,
        pltpu.SemaphoreType.DMA,
    ],
)
def _sc_stage(nf_hbm, src_hbm, dst_hbm, table_hbm, agg_out,
              nf_v, nid_v, x_chunk, src_v, dst_v, gidx_v, rows_v, zero_v,
              agg_sh, sem_in, sem_x, sem_rows, sem_sc):
    c = lax.axis_index("c")
    s = lax.axis_index("s")
    wid = s * NC + c
    base_n = wid * NODES_PER_W
    base_e = wid * NCHUNK

    cp_nf = pltpu.async_copy(nf_hbm, nf_v, sem_in)
    cp_src = pltpu.async_copy(src_hbm.at[pl.ds(base_e, NCHUNK)], src_v, sem_in)
    cp_dst = pltpu.async_copy(dst_hbm.at[pl.ds(base_e, NCHUNK)], dst_v, sem_in)

    def _zero_row(r, carry):
        zero_v[r, :] = jnp.zeros((LANES,), jnp.float32)
        return carry
    lax.fori_loop(0, ROWS_PER_SUB, _zero_row, 0)
    pltpu.sync_copy(zero_v, agg_sh.at[pl.ds(s * ROWS_PER_SUB, ROWS_PER_SUB)])

    for k in range(NODES_PER_W // LANES):
        nid_v[pl.ds(k * LANES, LANES)] = (
            base_n + k * LANES + lax.broadcasted_iota(jnp.int32, (LANES,), 0)
        )

    cp_nf.wait()
    cp_src.wait()
    cp_dst.wait()

    cp_x = pltpu.async_copy(
        table_hbm.at[nf_v.at[pl.ds(base_n, NODES_PER_W)]], x_chunk, sem_x
    )

    for j in range(NCHUNK):
        for i in range(CHUNK // LANES):
            sidx = src_v[j, pl.ds(i * LANES, LANES)]
            gidx_v[j, pl.ds(i * LANES, LANES)] = plsc.load_gather(nf_v, [sidx])

    row_cps = [
        pltpu.async_copy(
            table_hbm.at[gidx_v.at[j]],
            rows_v.at[pl.ds(j * CHUNK, CHUNK)],
            sem_rows,
        )
        for j in range(NCHUNK)
    ]

    plsc.subcore_barrier()

    cp_x.wait()
    sc_x = pltpu.async_copy(x_chunk, agg_sh.at[nid_v], sem_sc, add=True)
    for cp in row_cps:
        cp.wait()
    sc_cps = [
        pltpu.async_copy(
            rows_v.at[pl.ds(j * CHUNK, CHUNK)],
            agg_sh.at[dst_v.at[j]],
            sem_sc,
            add=True,
        )
        for j in range(NCHUNK)
    ]
    sc_x.wait()
    for cp in sc_cps:
        cp.wait()

    plsc.subcore_barrier()

    @pl.when(s == 0)
    def _():
        pltpu.sync_copy(agg_sh, agg_out.at[c])


BLK = 512
NBLK = N_NODES // BLK


def _tc_body(agg_ref, wei_ref, bei_ref, wat_ref, bat_ref, out_ref, ei_s):
    i = pl.program_id(0)

    @pl.when(i == 0)
    def _():
        h = agg_ref[0] + agg_ref[1]
        ei_s[...] = lax.dot_general(
            h, wei_ref[...], (((1,), (1,)), ((), ())),
            preferred_element_type=jnp.float32,
        ) + bei_ref[...]

    @pl.when(i < NBLK)
    def _():
        ei = ei_s[...]
        ei_blk = ei_s[pl.ds(i * BLK, BLK), :]
        gram = lax.dot_general(
            ei_blk, ei, (((1,), (1,)), ((), ())),
            preferred_element_type=jnp.float32,
        )
        out_ref[...] = gram.reshape(1, BLK * N_NODES)

    @pl.when(i == NBLK)
    def _():
        h0 = agg_ref[0, 0:1, :] + agg_ref[1, 0:1, :]
        at0 = lax.dot_general(
            h0, wat_ref[...], (((1,), (1,)), ((), ())),
            preferred_element_type=jnp.float32,
        ) + bat_ref[...]
        at = jnp.mean(at0, axis=-1, keepdims=True)
        out_ref[...] = jnp.broadcast_to(at, (1, BLK * N_NODES))


def _tc_stage(agg, W_ei, b_ei, W_at, b_at):
    return pl.pallas_call(
        _tc_body,
        grid=(NBLK + 1,),
        in_specs=[
            pl.BlockSpec((NC, N_NODES, EMB), lambda i: (0, 0, 0)),
            pl.BlockSpec((EMB, EMB), lambda i: (0, 0)),
            pl.BlockSpec((1, EMB), lambda i: (0, 0)),
            pl.BlockSpec((EMB, EMB), lambda i: (0, 0)),
            pl.BlockSpec((1, EMB), lambda i: (0, 0)),
        ],
        out_specs=pl.BlockSpec((1, BLK * N_NODES), lambda i: (0, i)),
        out_shape=jax.ShapeDtypeStruct((1, N_NODES * N_NODES + 1),
                                       jnp.float32),
        scratch_shapes=[pltpu.VMEM((N_NODES, EMB), jnp.float32)],
    )(agg, W_ei, b_ei, W_at, b_at)


def kernel(node_feature, batch_ptr, edge_index, node_index, batch_shape,
           emb_table, W_at, b_at, W_ei, b_ei):
    src = edge_index[:, 0].reshape(NW * NCHUNK, CHUNK)
    dst = edge_index[:, 1].reshape(NW * NCHUNK, CHUNK)
    agg = _sc_stage(node_feature, src, dst, emb_table)
    return _tc_stage(
        agg, W_ei, b_ei.reshape(1, EMB), W_at, b_at.reshape(1, EMB)
    )
